# baseline (speedup 1.0000x reference)
import functools

import jax
import jax.numpy as jnp
from jax import lax
from jax.experimental import pallas as pl
from jax.experimental.pallas import tpu as pltpu
from jax.experimental.pallas import tpu_sc as plsc

B = 4096
F = 26
C = 1000
OUT_W = F * C
NC, NS, L = 2, 16, 16
NW = NC * NS
NSLAB = 16
NHALF = 2
FH = F // NHALF
COLS = B // NSLAB
CROWS = C // 5
CH = C // CROWS
ITEMS = FH * CH
KV = COLS // L


def _sc_one_hot_t_body(xt_hbm, out_hbm, x_v, tile0, tile1, sem0, sem1):
    wid = lax.axis_index("s") * NC + lax.axis_index("c")
    slab = wid % NSLAB
    half = wid // NSLAB
    col0 = slab * COLS
    frow0 = half * FH * C

    pltpu.sync_copy(xt_hbm.at[:, pl.ds(col0, COLS)], x_v)

    lanes = lax.iota(jnp.int32, L)
    ones = jnp.full((L,), 1.0, jnp.float32)
    zeros = jnp.zeros((L,), jnp.float32)
    tiles = (tile0, tile1)
    sems = (sem0, sem1)

    def zbody(r, _):
        for k in range(KV):
            tile0[r, pl.ds(k * L, L)] = zeros
            tile1[r, pl.ds(k * L, L)] = zeros
        return 0
    lax.fori_loop(0, CROWS, zbody, 0)

    def item_fc(i):
        f = i // CH
        c0 = (i - CH * f) * CROWS
        return f, c0

    def dst(f, c0):
        r0 = pl.multiple_of(frow0 + f * C + c0, 8)
        return out_hbm.at[pl.ds(r0, CROWS), pl.ds(col0, COLS)]

    def sweep(tile, f, c0, val):
        for k in range(KV):
            bl = k * L + lanes
            xv = x_v[half * FH + f, pl.ds(k * L, L)]
            rel = xv - c0
            m = (rel >= 0) & (rel < CROWS)
            rel = jnp.minimum(jnp.maximum(rel, 0), CROWS - 1)
            plsc.store_scatter(tile, [rel, bl], val, mask=m)

    def paint_start(b, i):
        f, c0 = item_fc(i)
        sweep(tiles[b], f, c0, ones)
        pltpu.async_copy(tiles[b], dst(f, c0), sems[b])

    for b in range(2):
        paint_start(b, b)

    def body(j, _):
        for b in range(2):
            i = 2 * j + b
            f2, c02 = item_fc(i - 2)
            pltpu.make_async_copy(tiles[b], dst(f2, c02), sems[b]).wait()
            sweep(tiles[b], f2, c02, zeros)
            paint_start(b, i)
        return 0
    lax.fori_loop(1, ITEMS // 2, body, 0)

    for b in range(2):
        f2, c02 = item_fc(ITEMS - 2 + b)
        pltpu.make_async_copy(tiles[b], dst(f2, c02), sems[b]).wait()


_sc_one_hot_t = functools.partial(
    pl.kernel,
    out_type=jax.ShapeDtypeStruct((OUT_W, B), jnp.float32),
    mesh=plsc.VectorSubcoreMesh(core_axis_name="c", subcore_axis_name="s"),
    compiler_params=pltpu.CompilerParams(needs_layout_passes=False),
    scratch_types=[
        pltpu.VMEM((F, COLS), jnp.int32),
        pltpu.VMEM((CROWS, COLS), jnp.float32),
        pltpu.VMEM((CROWS, COLS), jnp.float32),
        pltpu.SemaphoreType.DMA,
        pltpu.SemaphoreType.DMA,
    ],
)(_sc_one_hot_t_body)


@jax.jit
def kernel(x):
    out_t = _sc_one_hot_t(x.astype(jnp.int32).T)
    return out_t.T


# speedup vs baseline: 1.0262x; 1.0262x over previous
import functools

import jax
import jax.numpy as jnp
from jax import lax
from jax.experimental import pallas as pl
from jax.experimental.pallas import tpu as pltpu
from jax.experimental.pallas import tpu_sc as plsc

B = 4096
F = 26
C = 1000
OUT_W = F * C
NC, NS, L = 2, 16, 16
NW = NC * NS
COLS = B // NW
CROWS = C // 5
CH = C // CROWS
ITEMS = F * CH
KV = COLS // L


def _sc_one_hot_t_body(xt_hbm, out_hbm, x_v, tile0, tile1, sem0, sem1):
    wid = lax.axis_index("s") * NC + lax.axis_index("c")
    col0 = wid * COLS

    pltpu.sync_copy(xt_hbm.at[:, pl.ds(col0, COLS)], x_v)

    lanes = lax.iota(jnp.int32, L)
    ones = jnp.full((L,), 1.0, jnp.float32)
    zeros = jnp.zeros((L,), jnp.float32)
    tiles = (tile0, tile1)
    sems = (sem0, sem1)

    def zbody(r, _):
        for k in range(KV):
            tile0[r, pl.ds(k * L, L)] = zeros
            tile1[r, pl.ds(k * L, L)] = zeros
        return 0
    lax.fori_loop(0, CROWS, zbody, 0)

    def item_fc(i):
        f = i // CH
        c0 = (i - CH * f) * CROWS
        return f, c0

    def dst(f, c0):
        return out_hbm.at[pl.ds(f * C + c0, CROWS), pl.ds(col0, COLS)]

    def sweep(tile, f, c0, val):
        for k in range(KV):
            bl = k * L + lanes
            xv = x_v[f, pl.ds(k * L, L)]
            rel = xv - c0
            m = (rel >= 0) & (rel < CROWS)
            rel = jnp.minimum(jnp.maximum(rel, 0), CROWS - 1)
            plsc.store_scatter(tile, [rel, bl], val, mask=m)

    def paint_start(b, i):
        f, c0 = item_fc(i)
        sweep(tiles[b], f, c0, ones)
        pltpu.async_copy(tiles[b], dst(f, c0), sems[b])

    for b in range(2):
        paint_start(b, b)

    def body(j, _):
        for b in range(2):
            i = 2 * j + b
            f2, c02 = item_fc(i - 2)
            pltpu.make_async_copy(tiles[b], dst(f2, c02), sems[b]).wait()
            sweep(tiles[b], f2, c02, zeros)
            paint_start(b, i)
        return 0
    lax.fori_loop(1, ITEMS // 2, body, 0)

    for b in range(2):
        f2, c02 = item_fc(ITEMS - 2 + b)
        pltpu.make_async_copy(tiles[b], dst(f2, c02), sems[b]).wait()


_sc_one_hot_t = functools.partial(
    pl.kernel,
    out_type=jax.ShapeDtypeStruct((OUT_W, B), jnp.float32),
    mesh=plsc.VectorSubcoreMesh(core_axis_name="c", subcore_axis_name="s"),
    compiler_params=pltpu.CompilerParams(needs_layout_passes=False),
    scratch_types=[
        pltpu.VMEM((F, COLS), jnp.int32),
        pltpu.VMEM((CROWS, COLS), jnp.float32),
        pltpu.VMEM((CROWS, COLS), jnp.float32),
        pltpu.SemaphoreType.DMA,
        pltpu.SemaphoreType.DMA,
    ],
)(_sc_one_hot_t_body)


@jax.jit
def kernel(x):
    out_t = _sc_one_hot_t(x.astype(jnp.int32).T)
    return out_t.T
